# Initial kernel scaffold; baseline (speedup 1.0000x reference)
import functools

import jax
import jax.numpy as jnp
from jax import lax
from jax.experimental import pallas as pl
from jax.experimental.pallas import tpu as pltpu
from jax.experimental.pallas import tpu_sc as plsc

NUM_NODES = 10000
NODE_NF = 128
GLOBAL_NF = 64
HIDDEN_NF = 256
NUM_GRAPHS = 64

NC = 2
NS = 16
NW = NC * NS
LANES = 16
COLB = NODE_NF // LANES

N_LO = 17
ROWS_LO = 320
ROWS_HI = 304


def _sc_segment_partials(x, batch_i32):
    mesh = plsc.VectorSubcoreMesh(core_axis_name="c", subcore_axis_name="s")

    @functools.partial(
        pl.kernel,
        mesh=mesh,
        out_type=[
            jax.ShapeDtypeStruct((NW, NUM_GRAPHS, NODE_NF), jnp.float32),
            jax.ShapeDtypeStruct((NW, NUM_GRAPHS), jnp.float32),
        ],
        scratch_types=[
            pltpu.VMEM((ROWS_LO, NODE_NF), jnp.float32),
            pltpu.VMEM((ROWS_LO,), jnp.int32),
            pltpu.VMEM((NUM_GRAPHS, NODE_NF), jnp.float32),
            pltpu.VMEM((NUM_GRAPHS,), jnp.float32),
        ],
    )
    def k(x_hbm, b_hbm, sums_hbm, cnts_hbm, xbuf, idxbuf, acc, cnt):
        cid = lax.axis_index("c")
        sid = lax.axis_index("s")
        wid = sid * NC + cid
        is_lo = wid < N_LO
        ngroups = jnp.where(is_lo, ROWS_LO // 16, ROWS_HI // 16)
        base_row = wid * ROWS_LO - 16 * jnp.maximum(wid - N_LO, 0)

        @pl.when(is_lo)
        def _():
            pltpu.sync_copy(x_hbm.at[pl.ds(base_row, ROWS_LO)], xbuf)
            pltpu.sync_copy(b_hbm.at[pl.ds(base_row, ROWS_LO)], idxbuf)

        @pl.when(jnp.logical_not(is_lo))
        def _():
            pltpu.sync_copy(x_hbm.at[pl.ds(base_row, ROWS_HI)],
                            xbuf.at[pl.ds(0, ROWS_HI)])
            pltpu.sync_copy(b_hbm.at[pl.ds(base_row, ROWS_HI)],
                            idxbuf.at[pl.ds(0, ROWS_HI)])

        zv = jnp.zeros((LANES,), jnp.float32)

        def zero_body(r, carry):
            for c in range(COLB):
                acc[r, pl.ds(c * LANES, LANES)] = zv
            return carry

        lax.fori_loop(0, NUM_GRAPHS, zero_body, 0)
        for c in range(NUM_GRAPHS // LANES):
            cnt[pl.ds(c * LANES, LANES)] = zv

        ones16 = jnp.ones((LANES,), jnp.float32)

        def group_body(g, carry):
            segs = idxbuf[pl.ds(g * 16, 16)]
            plsc.addupdate_scatter(cnt, [segs], ones16)

            def row_body(i, c2):
                r = g * 16 + i
                s = idxbuf[r]
                for c in range(COLB):
                    sl = pl.ds(c * LANES, LANES)
                    acc[s, sl] = acc[s, sl] + xbuf[r, sl]
                return c2

            lax.fori_loop(0, 16, row_body, 0)
            return carry

        lax.fori_loop(0, ngroups, group_body, 0)

        pltpu.sync_copy(acc, sums_hbm.at[wid])
        pltpu.sync_copy(cnt, cnts_hbm.at[wid])

    return k(x, batch_i32)


def _tc_head(psums, pcnts, u, w1u, w1m, b1, w2, b2):
    def body(ps_ref, pc_ref, u_ref, w1u_ref, w1m_ref, b1_ref, w2_ref, b2_ref,
             o_ref):
        sums = jnp.sum(ps_ref[...], axis=0)
        cnts = jnp.sum(pc_ref[...], axis=0)
        mean = sums / jnp.maximum(cnts, 1.0)[:, None]
        h = jnp.dot(u_ref[...], w1u_ref[...], preferred_element_type=jnp.float32)
        h = h + jnp.dot(mean, w1m_ref[...], preferred_element_type=jnp.float32)
        h = jnp.maximum(h + b1_ref[...], 0.0)
        o_ref[...] = (jnp.dot(h, w2_ref[...], preferred_element_type=jnp.float32)
                      + b2_ref[...])

    return pl.pallas_call(
        body,
        out_shape=jax.ShapeDtypeStruct((NUM_GRAPHS, GLOBAL_NF), jnp.float32),
    )(psums, pcnts, u, w1u, w1m, b1, w2, b2)


def kernel(x, edge_index, edge_attr, u, batch, W1, b1, W2, b2):
    batch_i32 = batch.astype(jnp.int32)
    psums, pcnts = _sc_segment_partials(x, batch_i32)
    w1u = W1[:GLOBAL_NF]
    w1m = W1[GLOBAL_NF:]
    return _tc_head(psums, pcnts, u, w1u, w1m,
                    b1.reshape(1, HIDDEN_NF), W2, b2.reshape(1, GLOBAL_NF))


# speedup vs baseline: 3.2436x; 3.2436x over previous
import functools

import jax
import jax.numpy as jnp
from jax import lax
from jax.experimental import pallas as pl
from jax.experimental.pallas import tpu as pltpu
from jax.experimental.pallas import tpu_sc as plsc

NUM_NODES = 10000
NODE_NF = 128
GLOBAL_NF = 64
HIDDEN_NF = 256
NUM_GRAPHS = 64

NC = 2
NS = 16
NW = NC * NS
LANES = 16
COLB = NODE_NF // LANES

N_LO = 17
ROWS_LO = 320
ROWS_HI = 304


def _sc_segment_partials(x, batch_i32):
    mesh = plsc.VectorSubcoreMesh(core_axis_name="c", subcore_axis_name="s")

    @functools.partial(
        pl.kernel,
        mesh=mesh,
        out_type=[
            jax.ShapeDtypeStruct((NW, NUM_GRAPHS, NODE_NF), jnp.float32),
            jax.ShapeDtypeStruct((NW, NUM_GRAPHS, LANES), jnp.float32),
        ],
        scratch_types=[
            pltpu.VMEM((ROWS_LO, NODE_NF), jnp.float32),
            pltpu.VMEM((ROWS_LO,), jnp.int32),
            pltpu.VMEM((NUM_GRAPHS, NODE_NF), jnp.float32),
            pltpu.VMEM((NUM_GRAPHS, LANES), jnp.float32),
        ],
    )
    def k(x_hbm, b_hbm, sums_hbm, cnts_hbm, xbuf, idxbuf, acc, cnt):
        cid = lax.axis_index("c")
        sid = lax.axis_index("s")
        wid = sid * NC + cid
        is_lo = wid < N_LO
        ngroups = jnp.where(is_lo, ROWS_LO // 16, ROWS_HI // 16)
        base_row = wid * ROWS_LO - 16 * jnp.maximum(wid - N_LO, 0)

        @pl.when(is_lo)
        def _():
            pltpu.sync_copy(x_hbm.at[pl.ds(base_row, ROWS_LO)], xbuf)
            pltpu.sync_copy(b_hbm.at[pl.ds(base_row, ROWS_LO)], idxbuf)

        @pl.when(jnp.logical_not(is_lo))
        def _():
            pltpu.sync_copy(x_hbm.at[pl.ds(base_row, ROWS_HI)],
                            xbuf.at[pl.ds(0, ROWS_HI)])
            pltpu.sync_copy(b_hbm.at[pl.ds(base_row, ROWS_HI)],
                            idxbuf.at[pl.ds(0, ROWS_HI)])

        zv = jnp.zeros((LANES,), jnp.float32)

        def zero_body(r, carry):
            for c in range(COLB):
                acc[r, pl.ds(c * LANES, LANES)] = zv
            cnt[r] = zv
            return carry

        lax.fori_loop(0, NUM_GRAPHS, zero_body, 0)

        ones16 = jnp.ones((LANES,), jnp.float32)

        def group_body(g, carry):
            segs = idxbuf[pl.ds(g * 16, 16)]
            for i in range(16):
                r = g * 16 + i
                s = segs[i]
                cnt[s] = cnt[s] + ones16
                for c in range(COLB):
                    sl = pl.ds(c * LANES, LANES)
                    acc[s, sl] = acc[s, sl] + xbuf[r, sl]
            return carry

        lax.fori_loop(0, ngroups, group_body, 0)

        pltpu.sync_copy(acc, sums_hbm.at[wid])
        pltpu.sync_copy(cnt, cnts_hbm.at[wid])

    return k(x, batch_i32)


def _tc_head(psums, pcnts, u, w1u, w1m, b1, w2, b2):
    def body(ps_ref, pc_ref, u_ref, w1u_ref, w1m_ref, b1_ref, w2_ref, b2_ref,
             o_ref):
        sums = jnp.sum(ps_ref[...], axis=0)
        cnts = jnp.sum(pc_ref[...], axis=0)[:, 0:1]
        mean = sums / jnp.maximum(cnts, 1.0)
        h = jnp.dot(u_ref[...], w1u_ref[...], preferred_element_type=jnp.float32)
        h = h + jnp.dot(mean, w1m_ref[...], preferred_element_type=jnp.float32)
        h = jnp.maximum(h + b1_ref[...], 0.0)
        o_ref[...] = (jnp.dot(h, w2_ref[...], preferred_element_type=jnp.float32)
                      + b2_ref[...])

    return pl.pallas_call(
        body,
        out_shape=jax.ShapeDtypeStruct((NUM_GRAPHS, GLOBAL_NF), jnp.float32),
    )(psums, pcnts, u, w1u, w1m, b1, w2, b2)


def kernel(x, edge_index, edge_attr, u, batch, W1, b1, W2, b2):
    batch_i32 = batch.astype(jnp.int32)
    psums, pcnts = _sc_segment_partials(x, batch_i32)
    w1u = W1[:GLOBAL_NF]
    w1m = W1[GLOBAL_NF:]
    return _tc_head(psums, pcnts, u, w1u, w1m,
                    b1.reshape(1, HIDDEN_NF), W2, b2.reshape(1, GLOBAL_NF))


# speedup vs baseline: 3.6603x; 1.1285x over previous
import functools

import jax
import jax.numpy as jnp
from jax import lax
from jax.experimental import pallas as pl
from jax.experimental.pallas import tpu as pltpu
from jax.experimental.pallas import tpu_sc as plsc

NUM_NODES = 10000
NODE_NF = 128
GLOBAL_NF = 64
HIDDEN_NF = 256
NUM_GRAPHS = 64

NC = 2
NS = 16
NW = NC * NS
LANES = 16
COLB = NODE_NF // LANES

N_LO = 17
ROWS_LO = 320
ROWS_HI = 304


def _sc_segment_partials(x, batch_i32):
    mesh = plsc.VectorSubcoreMesh(core_axis_name="c", subcore_axis_name="s")

    @functools.partial(
        pl.kernel,
        mesh=mesh,
        out_type=[
            jax.ShapeDtypeStruct((NW, NUM_GRAPHS, NODE_NF), jnp.float32),
            jax.ShapeDtypeStruct((NW, NUM_GRAPHS, LANES), jnp.float32),
        ],
        scratch_types=[
            pltpu.VMEM((ROWS_LO, NODE_NF), jnp.float32),
            pltpu.VMEM((ROWS_LO,), jnp.int32),
            pltpu.VMEM((NUM_GRAPHS, NODE_NF), jnp.float32),
            pltpu.VMEM((NUM_GRAPHS, LANES), jnp.float32),
            pltpu.VMEM((COLB + 1, LANES), jnp.float32),
        ],
    )
    def k(x_hbm, b_hbm, sums_hbm, cnts_hbm, xbuf, idxbuf, acc, cnt, areg):
        cid = lax.axis_index("c")
        sid = lax.axis_index("s")
        wid = sid * NC + cid
        is_lo = wid < N_LO
        ngroups = jnp.where(is_lo, ROWS_LO // 16, ROWS_HI // 16)
        base_row = wid * ROWS_LO - 16 * jnp.maximum(wid - N_LO, 0)

        @pl.when(is_lo)
        def _():
            pltpu.sync_copy(x_hbm.at[pl.ds(base_row, ROWS_LO)], xbuf)
            pltpu.sync_copy(b_hbm.at[pl.ds(base_row, ROWS_LO)], idxbuf)

        @pl.when(jnp.logical_not(is_lo))
        def _():
            pltpu.sync_copy(x_hbm.at[pl.ds(base_row, ROWS_HI)],
                            xbuf.at[pl.ds(0, ROWS_HI)])
            pltpu.sync_copy(b_hbm.at[pl.ds(base_row, ROWS_HI)],
                            idxbuf.at[pl.ds(0, ROWS_HI)])

        zv = jnp.zeros((LANES,), jnp.float32)

        def zero_body(r, carry):
            for c in range(COLB):
                acc[r, pl.ds(c * LANES, LANES)] = zv
            cnt[r] = zv
            return carry

        lax.fori_loop(0, NUM_GRAPHS, zero_body, 0)
        for c in range(COLB + 1):
            areg[c] = zv

        def flush_to_mem(cur):
            cnt[cur] = cnt[cur] + areg[COLB]
            for c in range(COLB):
                sl = pl.ds(c * LANES, LANES)
                acc[cur, sl] = acc[cur, sl] + areg[c]
            for c in range(COLB + 1):
                areg[c] = zv

        def group_body(g, cur):
            segs = idxbuf[pl.ds(g * 16, 16)]
            s0 = segs[0]
            s15 = segs[15]
            fast = jnp.logical_and(s0 == cur, s0 == s15)

            @pl.when(fast)
            def _():
                r0 = g * 16
                for c in range(COLB):
                    sl = pl.ds(c * LANES, LANES)
                    v = [xbuf[r0 + i, sl] for i in range(16)]
                    while len(v) > 1:
                        v = [v[2 * j] + v[2 * j + 1]
                             for j in range(len(v) // 2)]
                    areg[c] = areg[c] + v[0]
                areg[COLB] = areg[COLB] + 16.0

            @pl.when(jnp.logical_not(fast))
            def _():
                cur_ = cur
                for i in range(16):
                    s = segs[i]

                    @pl.when(s != cur_)
                    def _(cur_=cur_):
                        flush_to_mem(cur_)

                    r = g * 16 + i
                    for c in range(COLB):
                        sl = pl.ds(c * LANES, LANES)
                        areg[c] = areg[c] + xbuf[r, sl]
                    areg[COLB] = areg[COLB] + 1.0
                    cur_ = s

            return s15

        segs0 = idxbuf[pl.ds(0, 16)]
        fin = lax.fori_loop(0, ngroups, group_body, segs0[0])
        flush_to_mem(fin)

        pltpu.sync_copy(acc, sums_hbm.at[wid])
        pltpu.sync_copy(cnt, cnts_hbm.at[wid])

    return k(x, batch_i32)


def _tc_head(psums, pcnts, u, w1, b1, w2, b2):
    def body(ps_ref, pc_ref, u_ref, w1_ref, b1_ref, w2_ref, b2_ref, o_ref):
        sums = jnp.sum(ps_ref[...], axis=0)
        cnts = jnp.sum(pc_ref[...], axis=0)[:, 0:1]
        mean = sums / jnp.maximum(cnts, 1.0)
        w1u = w1_ref[0:GLOBAL_NF, :]
        w1m = w1_ref[GLOBAL_NF:, :]
        h = jnp.dot(u_ref[...], w1u, preferred_element_type=jnp.float32)
        h = h + jnp.dot(mean, w1m, preferred_element_type=jnp.float32)
        h = jnp.maximum(h + b1_ref[...], 0.0)
        o_ref[...] = (jnp.dot(h, w2_ref[...], preferred_element_type=jnp.float32)
                      + b2_ref[...])

    return pl.pallas_call(
        body,
        out_shape=jax.ShapeDtypeStruct((NUM_GRAPHS, GLOBAL_NF), jnp.float32),
    )(psums, pcnts, u, w1, b1, w2, b2)


def kernel(x, edge_index, edge_attr, u, batch, W1, b1, W2, b2):
    batch_i32 = batch.astype(jnp.int32)
    psums, pcnts = _sc_segment_partials(x, batch_i32)
    return _tc_head(psums, pcnts, u, W1,
                    b1.reshape(1, HIDDEN_NF), W2, b2.reshape(1, GLOBAL_NF))
